# Initial kernel scaffold; baseline (speedup 1.0000x reference)
import jax
import jax.numpy as jnp
from jax.experimental import pallas as pl


def kernel(x, edge_index, edge_attr, W1, b1, W2, b2, Wg1, bg1, Wg2, bg2, We1, be1, We2, be2):
    raise NotImplementedError("write your pallas kernel here")


# speedup vs baseline: 9.6284x; 9.6284x over previous
import functools

import jax
import jax.numpy as jnp
from jax import lax
from jax.experimental import pallas as pl
from jax.experimental.pallas import tpu as pltpu
from jax.experimental.pallas import tpu_sc as plsc

N = 100000
E = 1600000
H = 32

NC = 2
NS = 16
NW = NC * NS

C = 128
NCHUNK = E // C
N2 = N // 2
A_ROWS = 52224
ZROWS = 408
TRASH0 = N2

_mesh = plsc.VectorSubcoreMesh(core_axis_name="c", subcore_axis_name="s",
                               num_cores=NC, num_subcores=NS)
_sc_params = pltpu.CompilerParams(use_tc_tiling_on_sc=False)


@functools.partial(
    pl.kernel,
    out_type=(jax.ShapeDtypeStruct((N, 16), jnp.float32),
              jax.ShapeDtypeStruct((N, 16), jnp.float32)),
    mesh=_mesh,
    compiler_params=_sc_params,
    scratch_types=dict(
        hist=pltpu.VMEM_SHARED((N, 16), jnp.float32),
        zbuf=pltpu.VMEM((400, 16), jnp.float32),
        ones=pltpu.VMEM((C, 16), jnp.float32),
        dbuf=pltpu.VMEM((C,), jnp.int32),
    ),
)
def _deg_kernel(dst_hbm, cnt0_hbm, cnt1_hbm, hist, zbuf, ones, dbuf):
    c = lax.axis_index("c")
    s = lax.axis_index("s")
    wid = s * NC + c

    def fz(r, _):
        zbuf[r, pl.ds(0, 16)] = jnp.zeros((16,), jnp.float32)
        return 0
    lax.fori_loop(0, 400, fz, 0)

    def fo(r, _):
        ones[r, pl.ds(0, 16)] = jnp.ones((16,), jnp.float32)
        return 0
    lax.fori_loop(0, C, fo, 0)

    nz = jnp.where(s < NS - 1, 16, 10)

    def z(k, _):
        pltpu.sync_copy(zbuf, hist.at[pl.ds(s * 6400 + k * 400, 400)])
        return 0
    lax.fori_loop(0, nz, z, 0)

    plsc.subcore_barrier()

    nk = (NCHUNK - wid + NW - 1) // NW

    def chunk(k, _):
        base = (wid + k * NW) * C
        pltpu.sync_copy(dst_hbm.at[pl.ds(base, C)], dbuf)
        pltpu.sync_copy(ones, hist.at[dbuf], add=True)
        return 0
    lax.fori_loop(0, nk, chunk, 0)

    plsc.subcore_barrier()

    @pl.when(s < NS - 1)
    def _():
        @pl.when(c == 0)
        def _():
            pltpu.sync_copy(hist.at[pl.ds(s * 6400, 6400)],
                            cnt0_hbm.at[pl.ds(s * 6400, 6400)])

        @pl.when(c == 1)
        def _():
            pltpu.sync_copy(hist.at[pl.ds(s * 6400, 6400)],
                            cnt1_hbm.at[pl.ds(s * 6400, 6400)])

    @pl.when(s == NS - 1)
    def _():
        @pl.when(c == 0)
        def _():
            pltpu.sync_copy(hist.at[pl.ds(96000, 4000)],
                            cnt0_hbm.at[pl.ds(96000, 4000)])

        @pl.when(c == 1)
        def _():
            pltpu.sync_copy(hist.at[pl.ds(96000, 4000)],
                            cnt1_hbm.at[pl.ds(96000, 4000)])


@functools.partial(
    pl.kernel,
    out_type=jax.ShapeDtypeStruct((N, H), jnp.float32),
    mesh=_mesh,
    compiler_params=_sc_params,
    scratch_types=dict(
        acc=pltpu.VMEM_SHARED((A_ROWS, H), jnp.float32),
        zbuf=pltpu.VMEM((ZROWS, H), jnp.float32),
        sbuf=pltpu.VMEM((C,), jnp.int32),
        dbuf=pltpu.VMEM((C,), jnp.int32),
        ibuf=pltpu.VMEM((C,), jnp.int32),
        rows=pltpu.VMEM((C, H), jnp.float32),
        sem=pltpu.SemaphoreType.DMA,
    ),
)
def _scatter_kernel(src_hbm, dst_hbm, g_hbm, out_hbm,
                    acc, zbuf, sbuf, dbuf, ibuf, rows, sem):
    c = lax.axis_index("c")
    s = lax.axis_index("s")
    wid = s * NC + c
    lo = c * N2

    def fz(r, _):
        z = jnp.zeros((16,), jnp.float32)
        zbuf[r, pl.ds(0, 16)] = z
        zbuf[r, pl.ds(16, 16)] = z
        return 0
    lax.fori_loop(0, ZROWS, fz, 0)

    def z(k, _):
        pltpu.sync_copy(zbuf, acc.at[pl.ds(s * (8 * ZROWS) + k * ZROWS, ZROWS)])
        return 0
    lax.fori_loop(0, 8, z, 0)

    plsc.subcore_barrier()

    iota16 = jax.lax.iota(jnp.int32, 16)
    tbase = TRASH0 + s * C

    def chunk(k, _):
        base = (s + k * NS) * C
        ld = pltpu.async_copy(src_hbm.at[pl.ds(base, C)], sbuf, sem)
        ld2 = pltpu.async_copy(dst_hbm.at[pl.ds(base, C)], dbuf, sem)
        ld.wait()
        ld2.wait()

        def mk(j, _):
            d = dbuf[pl.ds(j * 16, 16)]
            rel = d - lo
            ok = (rel >= 0) & (rel < N2)
            ibuf[pl.ds(j * 16, 16)] = jnp.where(ok, rel, tbase + j * 16 + iota16)
            return 0
        lax.fori_loop(0, C // 16, mk, 0)

        pltpu.async_copy(g_hbm.at[sbuf], rows, sem).wait()
        pltpu.sync_copy(rows, acc.at[ibuf], add=True)
        return 0

    nk = (NCHUNK - s + NS - 1) // NS
    lax.fori_loop(0, nk, chunk, 0)

    plsc.subcore_barrier()

    @pl.when(s < NS - 1)
    def _():
        pltpu.sync_copy(acc.at[pl.ds(s * 3128, 3128)],
                        out_hbm.at[pl.ds(lo + s * 3128, 3128)])

    @pl.when(s == NS - 1)
    def _():
        pltpu.sync_copy(acc.at[pl.ds(46920, 3080)],
                        out_hbm.at[pl.ds(lo + 46920, 3080)])


@functools.partial(
    pl.kernel,
    out_type=jax.ShapeDtypeStruct((E, H), jnp.float32),
    mesh=_mesh,
    compiler_params=_sc_params,
    scratch_types=dict(
        sbuf=pltpu.VMEM((C,), jnp.int32),
        dbuf=pltpu.VMEM((C,), jnp.int32),
        prow=pltpu.VMEM((C, H), jnp.float32),
        qrow=pltpu.VMEM((C, H), jnp.float32),
        sem=pltpu.SemaphoreType.DMA,
    ),
)
def _edge_gather_kernel(src_hbm, dst_hbm, p_hbm, q_hbm, eo_hbm,
                        sbuf, dbuf, prow, qrow, sem):
    c = lax.axis_index("c")
    s = lax.axis_index("s")
    wid = s * NC + c

    def chunk(k, _):
        base = (wid + k * NW) * C
        ld = pltpu.async_copy(src_hbm.at[pl.ds(base, C)], sbuf, sem)
        ld2 = pltpu.async_copy(dst_hbm.at[pl.ds(base, C)], dbuf, sem)
        ld.wait()
        ld2.wait()
        gp = pltpu.async_copy(p_hbm.at[sbuf], prow, sem)
        gq = pltpu.async_copy(q_hbm.at[dbuf], qrow, sem)
        gp.wait()
        gq.wait()

        def add(r, _):
            prow[r, pl.ds(0, 16)] = prow[r, pl.ds(0, 16)] + qrow[r, pl.ds(0, 16)]
            prow[r, pl.ds(16, 16)] = prow[r, pl.ds(16, 16)] + qrow[r, pl.ds(16, 16)]
            return 0
        lax.fori_loop(0, C, add, 0)

        pltpu.sync_copy(prow, eo_hbm.at[pl.ds(base, C)])
        return 0

    nk = (NCHUNK - wid + NW - 1) // NW
    lax.fori_loop(0, nk, chunk, 0)


_BN = 2048
_BE = 6400


def _mm(w, a):
    return lax.dot_general(w, a, (((1,), (0,)), ((), ())),
                           preferred_element_type=jnp.float32)


def _node_a_body(xt_ref, cnt0t_ref, cnt1t_ref, w1_ref, b1_ref, w2_ref,
                 b2_ref, wg1_ref, g1_ref, dinv_ref):
    h = jnp.maximum(_mm(w1_ref[...], xt_ref[...]) + b1_ref[...][:, None], 0.0)
    h = _mm(w2_ref[...], h) + b2_ref[...][:, None]
    deg = cnt0t_ref[0:1, :] + cnt1t_ref[0:1, :] + 1.0
    dinv = lax.rsqrt(deg)
    dinv_ref[...] = dinv
    g1_ref[...] = _mm(wg1_ref[...], h) * dinv


def _node_b_body(tmp1t_ref, g1_ref, dinv_ref, bg1_ref, wg2_ref, g2_ref):
    dinv = dinv_ref[...]
    h1 = jnp.maximum(dinv * (tmp1t_ref[...] + g1_ref[...])
                     + bg1_ref[...][:, None], 0.0)
    g2_ref[...] = _mm(wg2_ref[...], h1) * dinv


def _node_c_body(tmp2t_ref, g2_ref, dinv_ref, bg2_ref, as_ref, ad_ref,
                 h_ref, p_ref, q_ref):
    h = jnp.maximum(dinv_ref[...] * (tmp2t_ref[...] + g2_ref[...])
                    + bg2_ref[...][:, None], 0.0)
    h_ref[...] = h
    p_ref[...] = _mm(as_ref[...], h)
    q_ref[...] = _mm(ad_ref[...], h)


def _edge_d_body(eot_ref, eat_ref, aattr_ref, be1_ref, we2_ref, be2_ref,
                 e_ref):
    t = eot_ref[...] + _mm(aattr_ref[...], eat_ref[...]) + be1_ref[...][:, None]
    t = jnp.maximum(t, 0.0)
    e_ref[...] = _mm(we2_ref[...], t) + be2_ref[...][:, None]


def _col_spec(rows, b):
    return pl.BlockSpec((rows, b), lambda i: (0, i))


def _full(shape):
    return pl.BlockSpec(shape, lambda i: tuple(0 for _ in shape))


def kernel(x, edge_index, edge_attr, W1, b1, W2, b2, Wg1, bg1, Wg2, bg2,
           We1, be1, We2, be2):
    src = edge_index[0]
    dst = edge_index[1]

    cnt0, cnt1 = _deg_kernel(dst)

    n_grid = pl.cdiv(N, _BN)

    g1t, dinv = pl.pallas_call(
        _node_a_body,
        grid=(n_grid,),
        in_specs=[_col_spec(2, _BN), _col_spec(16, _BN), _col_spec(16, _BN),
                  _full((H, 2)), _full((H,)), _full((H, H)), _full((H,)),
                  _full((H, H))],
        out_specs=[_col_spec(H, _BN), _col_spec(1, _BN)],
        out_shape=[jax.ShapeDtypeStruct((H, N), jnp.float32),
                   jax.ShapeDtypeStruct((1, N), jnp.float32)],
    )(x.T, cnt0.T, cnt1.T, W1, b1, W2, b2, Wg1)

    tmp1 = _scatter_kernel(src, dst, g1t.T)

    g2t = pl.pallas_call(
        _node_b_body,
        grid=(n_grid,),
        in_specs=[_col_spec(H, _BN), _col_spec(H, _BN), _col_spec(1, _BN),
                  _full((H,)), _full((H, H))],
        out_specs=_col_spec(H, _BN),
        out_shape=jax.ShapeDtypeStruct((H, N), jnp.float32),
    )(tmp1.T, g1t, dinv, bg1, Wg2)

    tmp2 = _scatter_kernel(src, dst, g2t.T)

    As = We1[:, :H]
    Ad = We1[:, H:2 * H]
    Aattr = We1[:, 2 * H:]

    ht, Pt, Qt = pl.pallas_call(
        _node_c_body,
        grid=(n_grid,),
        in_specs=[_col_spec(H, _BN), _col_spec(H, _BN), _col_spec(1, _BN),
                  _full((H,)), _full((H, H)), _full((H, H))],
        out_specs=[_col_spec(H, _BN)] * 3,
        out_shape=[jax.ShapeDtypeStruct((H, N), jnp.float32)] * 3,
    )(tmp2.T, g2t, dinv, bg2, As, Ad)

    eo = _edge_gather_kernel(src, dst, Pt.T, Qt.T)

    e_grid = E // _BE
    et = pl.pallas_call(
        _edge_d_body,
        grid=(e_grid,),
        in_specs=[_col_spec(H, _BE), _col_spec(3, _BE),
                  _full((H, 3)), _full((H,)), _full((H, H)), _full((H,))],
        out_specs=_col_spec(H, _BE),
        out_shape=jax.ShapeDtypeStruct((H, E), jnp.float32),
    )(eo.T, edge_attr.T, Aattr, be1, We2, be2)

    return ht.T, et.T
